# baseline (speedup 1.0000x reference)
import jax
import jax.numpy as jnp
from jax import lax
from jax.experimental import pallas as pl
from jax.experimental.pallas import tpu as pltpu
from jax.experimental.pallas import tpu_sc as plsc

_B, _S, _F, _D = 4, 8192, 6, 128
_NPOS = 1025
_L = 16
_NC, _NS = 2, 16
_NW = _NC * _NS
_TW = _B * _S // _NW
_CT = 64
_NK = _TW // _CT
_RING = 8


def _sc_body(bbox_hbm, x_hbm, y_hbm, h_hbm, w_hbm, out_hbm,
             x_sh, y_sh, h_sh, w_sh, idx_v, rows_v, gsem):
    cid = lax.axis_index("c")
    sid = lax.axis_index("s")
    wid = sid * _NC + cid
    bat = wid // (_S // _TW)
    s0 = (wid % (_S // _TW)) * _TW

    q = sid % 4
    tabs_hbm = (x_hbm, y_hbm, h_hbm, w_hbm)
    tabs_sh = (x_sh, y_sh, h_sh, w_sh)
    copies = []
    for f in range(4):
        @pl.when(sid // 4 == f)
        def _():
            pltpu.make_async_copy(
                tabs_hbm[f].at[pl.ds(q * 256, 256)],
                tabs_sh[f].at[pl.ds(q * 256, 256)],
                gsem,
            ).start()

            @pl.when(q == 3)
            def _():
                pltpu.make_async_copy(
                    tabs_hbm[f].at[pl.ds(1024, 1)],
                    tabs_sh[f].at[pl.ds(1024, 1)],
                    gsem,
                ).start()

    for f in range(_F):
        pltpu.make_async_copy(
            bbox_hbm.at[pl.ds(f * _B * _S + wid * _TW, _TW)],
            idx_v.at[pl.ds(f * _TW, _TW)],
            gsem,
        ).start()

    pltpu.make_async_copy(
        tabs_hbm[0].at[pl.ds(0, 256)], tabs_sh[0].at[pl.ds(0, 256)], gsem
    ).wait()

    @pl.when(q == 3)
    def _():
        pltpu.make_async_copy(
            tabs_hbm[0].at[pl.ds(1024, 1)], tabs_sh[0].at[pl.ds(1024, 1)], gsem
        ).wait()

    for f in range(_F):
        pltpu.make_async_copy(
            bbox_hbm.at[pl.ds(0, _TW)], idx_v.at[pl.ds(f * _TW, _TW)], gsem
        ).wait()

    plsc.subcore_barrier()

    fsrc = (x_sh, y_sh, x_sh, y_sh, h_sh, w_sh)

    def start_unit(k, f, b2):
        pltpu.make_async_copy(
            fsrc[f].at[idx_v.at[pl.ds(f * _TW + k * _CT, _CT)]],
            rows_v.at[b2],
            gsem,
        ).start()

    def wait_unit(b2):
        pltpu.make_async_copy(
            fsrc[0].at[idx_v.at[pl.ds(0, _CT)]],
            rows_v.at[b2],
            gsem,
        ).wait()

    for m in range(_RING - 1):
        start_unit(m // _F, m % _F, m % _RING)

    def chunk_quad(g, carry):
        k0 = 4 * g
        for j in range(4):
            k = k0 + j
            for f in range(_F):
                m6 = 6 * j + f
                b2 = m6 % _RING
                wait_unit(b2)
                fn = (f + _RING - 1) % 6
                kn = k + (f + _RING - 1) // 6

                @pl.when(kn < _NK)
                def _():
                    start_unit(kn, fn, (m6 + _RING - 1) % _RING)

                pltpu.sync_copy(
                    rows_v.at[b2],
                    out_hbm.at[bat, pl.ds(s0 + k * _CT, _CT),
                               pl.ds(f * _D, _D)],
                )
        return carry

    lax.fori_loop(0, _NK // 4, chunk_quad, 0)


@jax.jit
def _sc_embed(bbox_fm, x_tab, y_tab, h_tab, w_tab):
    mesh = plsc.VectorSubcoreMesh(core_axis_name="c", subcore_axis_name="s")
    return pl.kernel(
        _sc_body,
        mesh=mesh,
        compiler_params=pltpu.CompilerParams(needs_layout_passes=False),
        out_type=jax.ShapeDtypeStruct((_B, _S, _F * _D), jnp.float32),
        scratch_types=[
            pltpu.VMEM_SHARED((_NPOS, _D), jnp.float32),
            pltpu.VMEM_SHARED((_NPOS, _D), jnp.float32),
            pltpu.VMEM_SHARED((_NPOS, _D), jnp.float32),
            pltpu.VMEM_SHARED((_NPOS, _D), jnp.float32),
            pltpu.VMEM((_F * _TW,), jnp.int32),
            pltpu.VMEM((_RING, _CT, _D), jnp.float32),
            pltpu.SemaphoreType.DMA,
        ],
    )(bbox_fm, x_tab, y_tab, h_tab, w_tab)


def kernel(bbox, x_tab, y_tab, h_tab, w_tab):
    bbox_fm = bbox.astype(jnp.int32).transpose(2, 0, 1).reshape(_F * _B * _S)
    return _sc_embed(bbox_fm, x_tab, y_tab, h_tab, w_tab)


# speedup vs baseline: 1.0028x; 1.0028x over previous
import jax
import jax.numpy as jnp
from jax import lax
from jax.experimental import pallas as pl
from jax.experimental.pallas import tpu as pltpu
from jax.experimental.pallas import tpu_sc as plsc

_B, _S, _F, _D = 4, 8192, 6, 128
_NPOS = 1025
_L = 16
_NC, _NS = 2, 16
_NW = _NC * _NS
_TW = _B * _S // _NW
_CT = 128
_NK = _TW // _CT
_RING = 4


def _sc_body(bbox_hbm, x_hbm, y_hbm, h_hbm, w_hbm, out_hbm,
             x_sh, y_sh, h_sh, w_sh, idx_v, rows_v, gsem):
    cid = lax.axis_index("c")
    sid = lax.axis_index("s")
    wid = sid * _NC + cid
    bat = wid // (_S // _TW)
    s0 = (wid % (_S // _TW)) * _TW

    q = sid % 4
    tabs_hbm = (x_hbm, y_hbm, h_hbm, w_hbm)
    tabs_sh = (x_sh, y_sh, h_sh, w_sh)
    copies = []
    for f in range(4):
        @pl.when(sid // 4 == f)
        def _():
            pltpu.make_async_copy(
                tabs_hbm[f].at[pl.ds(q * 256, 256)],
                tabs_sh[f].at[pl.ds(q * 256, 256)],
                gsem,
            ).start()

            @pl.when(q == 3)
            def _():
                pltpu.make_async_copy(
                    tabs_hbm[f].at[pl.ds(1024, 1)],
                    tabs_sh[f].at[pl.ds(1024, 1)],
                    gsem,
                ).start()

    for f in range(_F):
        pltpu.make_async_copy(
            bbox_hbm.at[pl.ds(f * _B * _S + wid * _TW, _TW)],
            idx_v.at[pl.ds(f * _TW, _TW)],
            gsem,
        ).start()

    pltpu.make_async_copy(
        tabs_hbm[0].at[pl.ds(0, 256)], tabs_sh[0].at[pl.ds(0, 256)], gsem
    ).wait()

    @pl.when(q == 3)
    def _():
        pltpu.make_async_copy(
            tabs_hbm[0].at[pl.ds(1024, 1)], tabs_sh[0].at[pl.ds(1024, 1)], gsem
        ).wait()

    for f in range(_F):
        pltpu.make_async_copy(
            bbox_hbm.at[pl.ds(0, _TW)], idx_v.at[pl.ds(f * _TW, _TW)], gsem
        ).wait()

    plsc.subcore_barrier()

    fsrc = (x_sh, y_sh, x_sh, y_sh, h_sh, w_sh)

    def start_unit(k, f, b2):
        pltpu.make_async_copy(
            fsrc[f].at[idx_v.at[pl.ds(f * _TW + k * _CT, _CT)]],
            rows_v.at[b2],
            gsem,
        ).start()

    def wait_unit(b2):
        pltpu.make_async_copy(
            fsrc[0].at[idx_v.at[pl.ds(0, _CT)]],
            rows_v.at[b2],
            gsem,
        ).wait()

    for m in range(_RING - 1):
        start_unit(m // _F, m % _F, m % _RING)

    def chunk_quad(g, carry):
        k0 = 4 * g
        for j in range(4):
            k = k0 + j
            for f in range(_F):
                m6 = 6 * j + f
                b2 = m6 % _RING
                wait_unit(b2)
                fn = (f + _RING - 1) % 6
                kn = k + (f + _RING - 1) // 6

                @pl.when(kn < _NK)
                def _():
                    start_unit(kn, fn, (m6 + _RING - 1) % _RING)

                pltpu.sync_copy(
                    rows_v.at[b2],
                    out_hbm.at[bat, pl.ds(s0 + k * _CT, _CT),
                               pl.ds(f * _D, _D)],
                )
        return carry

    lax.fori_loop(0, _NK // 4, chunk_quad, 0)


@jax.jit
def _sc_embed(bbox_fm, x_tab, y_tab, h_tab, w_tab):
    mesh = plsc.VectorSubcoreMesh(core_axis_name="c", subcore_axis_name="s")
    return pl.kernel(
        _sc_body,
        mesh=mesh,
        compiler_params=pltpu.CompilerParams(needs_layout_passes=False),
        out_type=jax.ShapeDtypeStruct((_B, _S, _F * _D), jnp.float32),
        scratch_types=[
            pltpu.VMEM_SHARED((_NPOS, _D), jnp.float32),
            pltpu.VMEM_SHARED((_NPOS, _D), jnp.float32),
            pltpu.VMEM_SHARED((_NPOS, _D), jnp.float32),
            pltpu.VMEM_SHARED((_NPOS, _D), jnp.float32),
            pltpu.VMEM((_F * _TW,), jnp.int32),
            pltpu.VMEM((_RING, _CT, _D), jnp.float32),
            pltpu.SemaphoreType.DMA,
        ],
    )(bbox_fm, x_tab, y_tab, h_tab, w_tab)


def kernel(bbox, x_tab, y_tab, h_tab, w_tab):
    bbox_fm = bbox.astype(jnp.int32).transpose(2, 0, 1).reshape(_F * _B * _S)
    return _sc_embed(bbox_fm, x_tab, y_tab, h_tab, w_tab)
